# baseline (speedup 1.0000x reference)
import functools

import jax
import jax.numpy as jnp
from jax import lax
from jax.experimental import pallas as pl
from jax.experimental.pallas import tpu as pltpu
from jax.experimental.pallas import tpu_sc as plsc

_B = 128
_A = 64
_KNN = 8
_NT = 27
_NCH = 112
_HUGE = 0x7F800000
_MASKHI = ~2047


def _sc_body(cell_hbm, x_hbm, xt_hbm, out_hbm,
             cellb, xb, xtb, xcb, ub, scb, ssb, pN, edges, wbuf, minacc,
             outst):
    wid = lax.axis_index("s") * 2 + lax.axis_index("c")
    iota = lax.iota(jnp.int32, 16)
    i16 = iota * 16
    huge_i = jnp.full((16,), _HUGE, jnp.int32)

    def _vperm(v, idx):
        return lax.gather(
            v, idx[:, None],
            lax.GatherDimensionNumbers(offset_dims=(), collapsed_slice_dims=(0,),
                                       start_index_map=(0,)),
            (1,), mode=lax.GatherScatterMode.PROMISE_IN_BOUNDS)

    def _allsum(v):
        for sh in (8, 4, 2, 1):
            v = v + _vperm(v, iota ^ sh)
        return v

    for k in range(108, _NCH):
        pN[pl.ds(k * 16, 16)] = huge_i

    def structure_body(sl, outv):
        s = wid * 4 + sl
        pltpu.sync_copy(cell_hbm.at[s], cellb)
        pltpu.sync_copy(x_hbm.at[s], xb)
        pltpu.sync_copy(xt_hbm.at[s], xtb)

        cv = cellb[...]
        c = [[cv[3 * r + l] for l in range(3)] for r in range(3)]

        for cb in range(4):
            xcomp = [xb[pl.ds(comp * 64 + cb * 16, 16)] for comp in range(3)]
            xcc = [xcomp[0] * c[0][l] + xcomp[1] * c[1][l] + xcomp[2] * c[2][l]
                   for l in range(3)]
            for l in range(3):
                xcb[pl.ds(l * 64 + cb * 16, 16)] = xcc[l]
            ssb[pl.ds(cb * 16, 16)] = (
                xcc[0] * xcc[0] + xcc[1] * xcc[1] + xcc[2] * xcc[2])
            for comp in range(3):
                ub[pl.ds(comp * 64 + cb * 16, 16)] = (
                    xtb[pl.ds(comp * 64 + cb * 16, 16)] - xcomp[comp])

        for h in range(2):
            th = iota + 16 * h
            q3 = (th * 1366) >> 12
            q9 = (q3 * 1366) >> 12
            f0 = (q9 - 1).astype(jnp.float32)
            f1 = ((q3 % 3) - 1).astype(jnp.float32)
            f2 = ((th % 3) - 1).astype(jnp.float32)
            for l in range(3):
                scb[pl.ds(l * 32 + 16 * h, 16)] = (
                    f0 * c[0][l] + f1 * c[1][l] + f2 * c[2][l])

        bmn, bmx = [], []
        for l in range(3):
            vmn = xcb[pl.ds(l * 64, 16)]
            vmx = vmn
            for cb in range(1, 4):
                vv = xcb[pl.ds(l * 64 + cb * 16, 16)]
                vmn = jnp.minimum(vmn, vv)
                vmx = jnp.maximum(vmx, vv)
            svmn = lax.sort(vmn)
            svmx = lax.sort(vmx)
            bmn.append(svmn[0])
            bmx.append(svmx[15])

        def row_body(i, edgevec):
            xi = [xcb[pl.ds(comp * 64 + i, 16)][0] for comp in range(3)]
            selfidx = 13 * 64 + i
            xcv = [[xcb[pl.ds(comp * 64 + cb * 16, 16)] for cb in range(4)]
                   for comp in range(3)]

            m1 = huge_i
            m2 = huge_i
            for cb in range(4):
                tx = xcv[0][cb] - xi[0]
                ty = xcv[1][cb] - xi[1]
                tz = xcv[2][cb] - xi[2]
                d = tx * tx + ty * ty + tz * tz
                di = lax.bitcast_convert_type(d, jnp.int32)
                idxv = iota + (13 * 64 + cb * 16)
                key = (di & _MASKHI) | idxv
                key = jnp.where(idxv == selfidx, _HUGE, key)
                nm1 = jnp.minimum(m1, key)
                tt2 = jnp.maximum(m1, key)
                m2 = jnp.minimum(m2, tt2)
                m1 = nm1
                pN[pl.ds(13 * 64 + cb * 16, 16)] = key

            srt = lax.sort(m1)
            thrv = lax.bitcast_convert_type(srt & _MASKHI, jnp.float32)
            thr = thrv[7] * 1.001 + 1e-6

            def t_body(t, mm):
                tt = t + (t >= 13).astype(jnp.int32)
                z = [xi[comp] - scb[pl.ds(comp * 32 + tt, 16)][0]
                     for comp in range(3)]
                lb2 = jnp.float32(0.0)
                for l in range(3):
                    cl = jnp.minimum(jnp.maximum(z[l], bmn[l]), bmx[l])
                    dl = cl - z[l]
                    lb2 = lb2 + dl * dl
                t64 = tt * 64

                def _do(mm):
                    m1, m2 = mm
                    for cb in range(4):
                        tx = xcv[0][cb] - z[0]
                        ty = xcv[1][cb] - z[1]
                        tz = xcv[2][cb] - z[2]
                        d = tx * tx + ty * ty + tz * tz
                        di = lax.bitcast_convert_type(d, jnp.int32)
                        idxv = iota + (t64 + cb * 16)
                        key = (di & _MASKHI) | idxv
                        nm1 = jnp.minimum(m1, key)
                        tt2 = jnp.maximum(m1, key)
                        m2 = jnp.minimum(m2, tt2)
                        m1 = nm1
                        pN[pl.ds(t64 + cb * 16, 16)] = key
                    return m1, m2

                def _skip(mm):
                    for cb in range(4):
                        pN[pl.ds(t64 + cb * 16, 16)] = huge_i
                    return mm

                return lax.cond(lb2 <= thr, _do, _skip, mm)

            min1v, min2v = lax.fori_loop(0, 26, t_body, (m1, m2))

            half = 8 * (i & 1)
            for e in range(_KNN):
                pm = lax.sort(min1v)[0]
                idx = pm & 2047
                lane = pm & 15
                edgevec = jnp.where(iota == half + e, idx, edgevec)
                promoted = _vperm(min2v, jnp.full((16,), lane, jnp.int32))
                sec = promoted[0]
                min1v = jnp.where(iota == lane, promoted, min1v)
                min2v = jnp.where(iota == lane, _HUGE, min2v)

                def _repair(mm, lane=lane, pm=pm):
                    m1, m2 = mm
                    a = huge_i
                    b = huge_i
                    for k in range(7):
                        g = plsc.load_gather(pN, [i16 + (lane + 256 * k)])
                        g = jnp.where(g > pm, g, _HUGE)
                        na = jnp.minimum(a, g)
                        tt = jnp.maximum(a, g)
                        b = jnp.minimum(b, tt)
                        a = na
                    s1 = lax.sort(a)[0]
                    a2 = jnp.where(a == s1, b, a)
                    s2 = lax.sort(a2)[0]
                    m1 = jnp.where(iota == lane, s1, m1)
                    m2 = jnp.where(iota == lane, s2, m2)
                    return m1, m2

                min1v, min2v = lax.cond(sec == _HUGE, _repair,
                                        lambda mm: mm, (min1v, min2v))

            edges[pl.ds((i >> 1) * 16, 16)] = edgevec
            return edgevec

        lax.fori_loop(0, _A, row_body, jnp.zeros((16,), jnp.int32))

        def g_body(g, _):
            ev = edges[pl.ds(g * 16, 16)]
            j = ev & 63
            ivec = 2 * g + (iota >> 3)
            uj = [plsc.load_gather(ub, [j + comp * 64]) for comp in range(3)]
            ui = [plsc.load_gather(ub, [ivec + comp * 64]) for comp in range(3)]
            dx = uj[0] - ui[0]
            dy = uj[1] - ui[1]
            dz = uj[2] - ui[2]
            for l in range(3):
                wbuf[pl.ds(l * 512 + g * 16, 16)] = (
                    dx * c[0][l] + dy * c[1][l] + dz * c[2][l])
            minacc[pl.ds(g * 16, 16)] = jnp.full((16,), 1e30, jnp.float32)
            return 0

        lax.fori_loop(0, 32, g_body, 0)

        def r_body(r, _):
            scr = [scb[pl.ds(l * 32 + r, 16)][0] for l in range(3)]

            def g2_body(g, _):
                sl16 = pl.ds(g * 16, 16)
                ax = wbuf[pl.ds(0 * 512 + g * 16, 16)] + scr[0]
                ay = wbuf[pl.ds(1 * 512 + g * 16, 16)] + scr[1]
                az = wbuf[pl.ds(2 * 512 + g * 16, 16)] + scr[2]
                n = ax * ax + ay * ay + az * az
                minacc[sl16] = jnp.minimum(minacc[sl16], n)
                return 0

            lax.fori_loop(0, 32, g2_body, 0)
            return 0

        lax.fori_loop(0, _NT, r_body, 0)

        def s_body(g, acc):
            n = minacc[pl.ds(g * 16, 16)] + 1e-12
            ni = lax.bitcast_convert_type(n, jnp.int32)
            y = lax.bitcast_convert_type(0x5F3759DF - (ni >> 1), jnp.float32)
            hn = 0.5 * n
            for _it in range(3):
                y = y * (1.5 - hn * y * y)
            return acc + n * y

        acc = lax.fori_loop(0, 32, s_body, jnp.zeros((16,), jnp.float32))
        totv = _allsum(acc) * jnp.float32(1.0 / 512.0)
        return jnp.where(iota == sl, totv, outv)

    outv = lax.fori_loop(0, 4, structure_body, jnp.zeros((16,), jnp.float32))
    outst[...] = outv
    pltpu.sync_copy(outst, out_hbm.at[wid])


_sc_call = functools.partial(
    pl.kernel,
    out_type=jax.ShapeDtypeStruct((32, 16), jnp.float32),
    mesh=plsc.VectorSubcoreMesh(core_axis_name="c", subcore_axis_name="s"),
    compiler_params=pltpu.CompilerParams(needs_layout_passes=False),
    scratch_types=[
        pltpu.VMEM((16,), jnp.float32),
        pltpu.VMEM((192,), jnp.float32),
        pltpu.VMEM((192,), jnp.float32),
        pltpu.VMEM((224,), jnp.float32),
        pltpu.VMEM((192,), jnp.float32),
        pltpu.VMEM((128,), jnp.float32),
        pltpu.VMEM((64,), jnp.float32),
        pltpu.VMEM((16 * _NCH,), jnp.int32),
        pltpu.VMEM((512,), jnp.int32),
        pltpu.VMEM((1536,), jnp.float32),
        pltpu.VMEM((512,), jnp.float32),
        pltpu.VMEM((16,), jnp.float32),
    ],
)(_sc_body)


def kernel(cell, x, x_tilde, num_atoms):
    del num_atoms
    cell_p = jnp.concatenate(
        [cell.reshape(_B, 9), jnp.zeros((_B, 7), jnp.float32)], axis=1)
    xp = x.reshape(_B, _A, 3).transpose(0, 2, 1).reshape(_B, 192)
    xtp = x_tilde.reshape(_B, _A, 3).transpose(0, 2, 1).reshape(_B, 192)
    out = _sc_call(cell_p, xp, xtp)
    return out[:, :4].reshape(_B)


# speedup vs baseline: 1.1834x; 1.1834x over previous
import functools

import jax
import jax.numpy as jnp
from jax import lax
from jax.experimental import pallas as pl
from jax.experimental.pallas import tpu as pltpu
from jax.experimental.pallas import tpu_sc as plsc

_B = 128
_A = 64
_KNN = 8
_NT = 27
_NCH = 112
_HUGE = 0x7F800000
_MASKHI = ~2047


def _sc_body(cell_hbm, x_hbm, xt_hbm, out_hbm,
             cellb, xb, xtb, xcb, ub, scb, ssb, pN, edges, wbuf, minacc,
             outst):
    wid = lax.axis_index("s") * 2 + lax.axis_index("c")
    iota = lax.iota(jnp.int32, 16)
    i16 = iota * 16
    huge_i = jnp.full((16,), _HUGE, jnp.int32)

    def _vperm(v, idx):
        return lax.gather(
            v, idx[:, None],
            lax.GatherDimensionNumbers(offset_dims=(), collapsed_slice_dims=(0,),
                                       start_index_map=(0,)),
            (1,), mode=lax.GatherScatterMode.PROMISE_IN_BOUNDS)

    def _allsum(v):
        for sh in (8, 4, 2, 1):
            v = v + _vperm(v, iota ^ sh)
        return v

    for k in range(108, _NCH):
        pN[pl.ds(k * 16, 16)] = huge_i

    def structure_body(sl, outv):
        s = wid * 4 + sl
        pltpu.sync_copy(cell_hbm.at[s], cellb)
        pltpu.sync_copy(x_hbm.at[s], xb)
        pltpu.sync_copy(xt_hbm.at[s], xtb)

        cv = cellb[...]
        c = [[cv[3 * r + l] for l in range(3)] for r in range(3)]

        for cb in range(4):
            xcomp = [xb[pl.ds(comp * 64 + cb * 16, 16)] for comp in range(3)]
            xcc = [xcomp[0] * c[0][l] + xcomp[1] * c[1][l] + xcomp[2] * c[2][l]
                   for l in range(3)]
            for l in range(3):
                xcb[pl.ds(l * 64 + cb * 16, 16)] = xcc[l]
            ssb[pl.ds(cb * 16, 16)] = (
                xcc[0] * xcc[0] + xcc[1] * xcc[1] + xcc[2] * xcc[2])
            for comp in range(3):
                ub[pl.ds(comp * 64 + cb * 16, 16)] = (
                    xtb[pl.ds(comp * 64 + cb * 16, 16)] - xcomp[comp])

        for h in range(2):
            th = iota + 16 * h
            q3 = (th * 1366) >> 12
            q9 = (q3 * 1366) >> 12
            f0 = (q9 - 1).astype(jnp.float32)
            f1 = ((q3 % 3) - 1).astype(jnp.float32)
            f2 = ((th % 3) - 1).astype(jnp.float32)
            for l in range(3):
                scb[pl.ds(l * 32 + 16 * h, 16)] = (
                    f0 * c[0][l] + f1 * c[1][l] + f2 * c[2][l])

        def row_body(i, edgevec):
            xi = [xcb[pl.ds(comp * 64 + i, 16)][0] for comp in range(3)]
            selfidx = 13 * 64 + i
            xcv = [[xcb[pl.ds(comp * 64 + cb * 16, 16)] for cb in range(4)]
                   for comp in range(3)]

            def t_body(t, mm):
                m1, m2 = mm
                z = [xi[comp] - scb[pl.ds(comp * 32 + t, 16)][0]
                     for comp in range(3)]
                t64 = t * 64
                for cb in range(4):
                    tx = xcv[0][cb] - z[0]
                    ty = xcv[1][cb] - z[1]
                    tz = xcv[2][cb] - z[2]
                    d = tx * tx + ty * ty + tz * tz
                    di = lax.bitcast_convert_type(d, jnp.int32)
                    idxv = iota + (t64 + cb * 16)
                    key = (di & _MASKHI) | idxv
                    key = jnp.where(idxv == selfidx, _HUGE, key)
                    nm1 = jnp.minimum(m1, key)
                    tt2 = jnp.maximum(m1, key)
                    m2 = jnp.minimum(m2, tt2)
                    m1 = nm1
                    pN[pl.ds(t64 + cb * 16, 16)] = key
                return m1, m2

            min1v, min2v = lax.fori_loop(0, _NT, t_body, (huge_i, huge_i))

            half = 8 * (i & 1)
            for e in range(_KNN):
                pm = lax.sort(min1v)[0]
                idx = pm & 2047
                lane = pm & 15
                edgevec = jnp.where(iota == half + e, idx, edgevec)
                promoted = _vperm(min2v, jnp.full((16,), lane, jnp.int32))
                sec = promoted[0]
                min1v = jnp.where(iota == lane, promoted, min1v)
                min2v = jnp.where(iota == lane, _HUGE, min2v)

                def _repair(mm, lane=lane, pm=pm):
                    m1, m2 = mm
                    a = huge_i
                    b = huge_i
                    for k in range(7):
                        g = plsc.load_gather(pN, [i16 + (lane + 256 * k)])
                        g = jnp.where(g > pm, g, _HUGE)
                        na = jnp.minimum(a, g)
                        tt = jnp.maximum(a, g)
                        b = jnp.minimum(b, tt)
                        a = na
                    s1 = lax.sort(a)[0]
                    a2 = jnp.where(a == s1, b, a)
                    s2 = lax.sort(a2)[0]
                    m1 = jnp.where(iota == lane, s1, m1)
                    m2 = jnp.where(iota == lane, s2, m2)
                    return m1, m2

                min1v, min2v = lax.cond(sec == _HUGE, _repair,
                                        lambda mm: mm, (min1v, min2v))

            edges[pl.ds((i >> 1) * 16, 16)] = edgevec
            return edgevec

        lax.fori_loop(0, _A, row_body, jnp.zeros((16,), jnp.int32))

        def g_body(g, _):
            ev = edges[pl.ds(g * 16, 16)]
            j = ev & 63
            ivec = 2 * g + (iota >> 3)
            uj = [plsc.load_gather(ub, [j + comp * 64]) for comp in range(3)]
            ui = [plsc.load_gather(ub, [ivec + comp * 64]) for comp in range(3)]
            dx = uj[0] - ui[0]
            dy = uj[1] - ui[1]
            dz = uj[2] - ui[2]
            for l in range(3):
                wbuf[pl.ds(l * 512 + g * 16, 16)] = (
                    dx * c[0][l] + dy * c[1][l] + dz * c[2][l])
            minacc[pl.ds(g * 16, 16)] = jnp.full((16,), 1e30, jnp.float32)
            return 0

        lax.fori_loop(0, 32, g_body, 0)

        def r_body(r, _):
            scr = [scb[pl.ds(l * 32 + r, 16)][0] for l in range(3)]

            def g2_body(g, _):
                sl16 = pl.ds(g * 16, 16)
                ax = wbuf[pl.ds(0 * 512 + g * 16, 16)] + scr[0]
                ay = wbuf[pl.ds(1 * 512 + g * 16, 16)] + scr[1]
                az = wbuf[pl.ds(2 * 512 + g * 16, 16)] + scr[2]
                n = ax * ax + ay * ay + az * az
                minacc[sl16] = jnp.minimum(minacc[sl16], n)
                return 0

            lax.fori_loop(0, 32, g2_body, 0)
            return 0

        lax.fori_loop(0, _NT, r_body, 0)

        def s_body(g, acc):
            n = minacc[pl.ds(g * 16, 16)] + 1e-12
            ni = lax.bitcast_convert_type(n, jnp.int32)
            y = lax.bitcast_convert_type(0x5F3759DF - (ni >> 1), jnp.float32)
            hn = 0.5 * n
            for _it in range(3):
                y = y * (1.5 - hn * y * y)
            return acc + n * y

        acc = lax.fori_loop(0, 32, s_body, jnp.zeros((16,), jnp.float32))
        totv = _allsum(acc) * jnp.float32(1.0 / 512.0)
        return jnp.where(iota == sl, totv, outv)

    outv = lax.fori_loop(0, 4, structure_body, jnp.zeros((16,), jnp.float32))
    outst[...] = outv
    pltpu.sync_copy(outst, out_hbm.at[wid])


_sc_call = functools.partial(
    pl.kernel,
    out_type=jax.ShapeDtypeStruct((32, 16), jnp.float32),
    mesh=plsc.VectorSubcoreMesh(core_axis_name="c", subcore_axis_name="s"),
    compiler_params=pltpu.CompilerParams(needs_layout_passes=False),
    scratch_types=[
        pltpu.VMEM((16,), jnp.float32),
        pltpu.VMEM((192,), jnp.float32),
        pltpu.VMEM((192,), jnp.float32),
        pltpu.VMEM((224,), jnp.float32),
        pltpu.VMEM((192,), jnp.float32),
        pltpu.VMEM((128,), jnp.float32),
        pltpu.VMEM((64,), jnp.float32),
        pltpu.VMEM((16 * _NCH,), jnp.int32),
        pltpu.VMEM((512,), jnp.int32),
        pltpu.VMEM((1536,), jnp.float32),
        pltpu.VMEM((512,), jnp.float32),
        pltpu.VMEM((16,), jnp.float32),
    ],
)(_sc_body)


def kernel(cell, x, x_tilde, num_atoms):
    del num_atoms
    cell_p = jnp.concatenate(
        [cell.reshape(_B, 9), jnp.zeros((_B, 7), jnp.float32)], axis=1)
    xp = x.reshape(_B, _A, 3).transpose(0, 2, 1).reshape(_B, 192)
    xtp = x_tilde.reshape(_B, _A, 3).transpose(0, 2, 1).reshape(_B, 192)
    out = _sc_call(cell_p, xp, xtp)
    return out[:, :4].reshape(_B)
